# baseline (speedup 1.0000x reference)
import jax
import jax.numpy as jnp
from jax import lax
from jax.experimental import pallas as pl
from jax.experimental.pallas import tpu as pltpu
from jax.experimental.pallas import tpu_sc as plsc

_NC, _NS = 2, 16
_NW = _NC * _NS


def _splat16(x):
    return lax.broadcast(x, (16,))


def _make_sc_kernel(B, T, T_vis, D_vis, T_prop, D_prop, L, D_lang):
    D_out = D_vis + D_prop + D_lang
    SPW = B // _NW
    CH = 32
    NCH = T // CH
    LCH = 8
    inv_L = 1.0 / L

    def body(vis_f, vis_t, prop_f, prop_t, lang, tgt, out,
             tgt_v, vist_v, propt_v,
             visidxL_v, visidxR_v, visw_v, propidx_v, propw_v,
             propfeat_v, langsum_v, langstage_v, fused_v,
             visL_v, visR_v, sem):
        wid = lax.axis_index("s") * _NC + lax.axis_index("c")

        def searchsorted(track_v, T_src, n_steps, base_row,
                         idxL_v, idxR_v, w_v):
            def chunk(i, carry):
                t16 = tgt_v[pl.ds(i * 16, 16)]
                lo = jnp.zeros((16,), jnp.int32)
                hi = jnp.full((16,), T_src, jnp.int32)
                for _ in range(n_steps):
                    mid = jnp.minimum(lax.shift_right_logical(lo + hi, 1),
                                      T_src - 1)
                    tm = plsc.load_gather(track_v, [mid])
                    pred = tm < t16
                    lo = jnp.where(pred, mid + 1, lo)
                    hi = jnp.where(pred, hi, mid)
                idx = jnp.minimum(lo, T_src - 2)
                tl = plsc.load_gather(track_v, [idx])
                tr = plsc.load_gather(track_v, [idx + 1])
                w = jnp.clip((t16 - tl) / (tr - tl + 1e-8), 0.0, 1.0)
                idxL_v[pl.ds(i * 16, 16)] = idx + base_row
                if idxR_v is not None:
                    idxR_v[pl.ds(i * 16, 16)] = idx + 1 + base_row
                w_v[pl.ds(i * 16, 16)] = w
                return carry
            lax.fori_loop(0, T // 16, chunk, 0)

        def sample(s, carry):
            b = wid * SPW + s
            pltpu.sync_copy(tgt.at[b], tgt_v)
            pltpu.sync_copy(vis_t.at[b], vist_v)
            pltpu.sync_copy(prop_t.at[b], propt_v)
            pltpu.sync_copy(prop_f.at[b], propfeat_v)

            searchsorted(vist_v, T_vis, 8, b * T_vis,
                         visidxL_v, visidxR_v, visw_v)
            searchsorted(propt_v, T_prop, 10, 0,
                         propidx_v, None, propw_v)

            def zero(c, carry):
                langsum_v[pl.ds(c * 16, 16)] = jnp.zeros((16,), jnp.float32)
                return carry
            lax.fori_loop(0, D_lang // 16, zero, 0)
            row0 = 0
            while row0 < L:
                rows = min(LCH, L - row0)
                pltpu.sync_copy(lang.at[b, pl.ds(row0, rows), :],
                                langstage_v.at[pl.ds(0, rows), :])

                def acc(r, carry):
                    for c in range(D_lang // 16):
                        sl = pl.ds(c * 16, 16)
                        langsum_v[sl] = langsum_v[sl] + langstage_v[r, sl]
                    return carry
                lax.fori_loop(0, rows, acc, 0)
                row0 += rows

            def scale(c, carry):
                sl = pl.ds(c * 16, 16)
                langsum_v[sl] = langsum_v[sl] * inv_L
                return carry
            lax.fori_loop(0, D_lang // 16, scale, 0)

            def fill(r, carry):
                for c in range(D_lang // 16):
                    fused_v[r, pl.ds(D_vis + D_prop + c * 16, 16)] = (
                        langsum_v[pl.ds(c * 16, 16)])
                return carry
            lax.fori_loop(0, CH, fill, 0)

            lane = lax.iota(jnp.int32, 16)
            for k in range(NCH):
                cvL = pltpu.async_copy(
                    vis_f.at[visidxL_v.at[pl.ds(k * CH, CH)]], visL_v, sem)
                cvR = pltpu.async_copy(
                    vis_f.at[visidxR_v.at[pl.ds(k * CH, CH)]], visR_v, sem)
                cvL.wait()
                cvR.wait()

                def lerp(r, carry):
                    g = _splat16(k * CH + r)
                    wv = plsc.load_gather(visw_v, [g])
                    for c in range(D_vis // 16):
                        sl = pl.ds(c * 16, 16)
                        lv = visL_v[r, sl]
                        rv = visR_v[r, sl]
                        fused_v[r, sl] = lv + wv * (rv - lv)
                    wp = plsc.load_gather(propw_v, [g])
                    pi = plsc.load_gather(propidx_v, [g])
                    for c in range(D_prop // 16):
                        col = lane + c * 16
                        lv = plsc.load_gather(propfeat_v, [pi, col])
                        rv = plsc.load_gather(propfeat_v, [pi + 1, col])
                        fused_v[r, pl.ds(D_vis + c * 16, 16)] = (
                            lv + wp * (rv - lv))
                    return carry
                lax.fori_loop(0, CH, lerp, 0)

                pltpu.sync_copy(fused_v, out.at[b, pl.ds(k * CH, CH), :])
            return carry

        lax.fori_loop(0, SPW, sample, 0)

    mesh = plsc.VectorSubcoreMesh(core_axis_name="c", subcore_axis_name="s")
    return pl.kernel(
        body,
        out_type=jax.ShapeDtypeStruct((B, T, D_out), jnp.float32),
        mesh=mesh,
        compiler_params=pltpu.CompilerParams(needs_layout_passes=False),
        scratch_types=[
            pltpu.VMEM((T,), jnp.float32),
            pltpu.VMEM((T_vis,), jnp.float32),
            pltpu.VMEM((T_prop,), jnp.float32),
            pltpu.VMEM((T,), jnp.int32),
            pltpu.VMEM((T,), jnp.int32),
            pltpu.VMEM((T,), jnp.float32),
            pltpu.VMEM((T,), jnp.int32),
            pltpu.VMEM((T,), jnp.float32),
            pltpu.VMEM((T_prop, D_prop), jnp.float32),
            pltpu.VMEM((D_lang,), jnp.float32),
            pltpu.VMEM((LCH, D_lang), jnp.float32),
            pltpu.VMEM((CH, D_out), jnp.float32),
            pltpu.VMEM((CH, D_vis), jnp.float32),
            pltpu.VMEM((CH, D_vis), jnp.float32),
            pltpu.SemaphoreType.DMA,
        ],
    )


def kernel(vision_features, vision_timestamps, proprio_features,
           proprio_timestamps, lang_embeddings, target_timestamps):
    B, T_vis, D_vis = vision_features.shape
    _, T_prop, D_prop = proprio_features.shape
    _, L, D_lang = lang_embeddings.shape
    T = target_timestamps.shape[1]

    k = _make_sc_kernel(B, T, T_vis, D_vis, T_prop, D_prop, L, D_lang)
    return k(vision_features.reshape(B * T_vis, D_vis), vision_timestamps,
             proprio_features, proprio_timestamps, lang_embeddings,
             target_timestamps)


# speedup vs baseline: 1.6597x; 1.6597x over previous
import jax
import jax.numpy as jnp
from jax import lax
from jax.experimental import pallas as pl
from jax.experimental.pallas import tpu as pltpu
from jax.experimental.pallas import tpu_sc as plsc

_NC, _NS = 2, 16
_NW = _NC * _NS


def _splat16(x):
    return lax.broadcast(x, (16,))


def _lang_mean_kernel(lang_ref, out_ref):
    out_ref[...] = jnp.mean(lang_ref[...], axis=1)


def _lang_mean(lang_embeddings):
    B, L, D_lang = lang_embeddings.shape
    BB = 8
    return pl.pallas_call(
        _lang_mean_kernel,
        grid=(B // BB,),
        in_specs=[pl.BlockSpec((BB, L, D_lang), lambda i: (i, 0, 0))],
        out_specs=pl.BlockSpec((BB, D_lang), lambda i: (i, 0)),
        out_shape=jax.ShapeDtypeStruct((B, D_lang), jnp.float32),
    )(lang_embeddings)


def _make_sc_kernel(B, T, T_vis, D_vis, T_prop, D_prop, D_lang):
    D_out = D_vis + D_prop + D_lang
    SPW = B // _NW
    CH = 16
    NCH = T // CH

    def body(vis_f, vis_t, prop_f, prop_t, lang_avg, tgt, out,
             tgt_v, vist_v, propt_v,
             visidxL_v, visidxR_v, visw_v, propidx_v, propw_v,
             propfeat_v, langsum_v,
             fused0_v, fused1_v, visL0_v, visL1_v, visR0_v, visR1_v,
             gsem, osem):
        wid = lax.axis_index("s") * _NC + lax.axis_index("c")
        fused = (fused0_v, fused1_v)
        visL = (visL0_v, visL1_v)
        visR = (visR0_v, visR1_v)

        def searchsorted(track_v, T_src, n_steps, base_row,
                         idxL_v, idxR_v, w_v):
            def chunk(i, carry):
                t16 = tgt_v[pl.ds(i * 16, 16)]
                lo = jnp.zeros((16,), jnp.int32)
                hi = jnp.full((16,), T_src, jnp.int32)
                for _ in range(n_steps):
                    mid = jnp.minimum(lax.shift_right_logical(lo + hi, 1),
                                      T_src - 1)
                    tm = plsc.load_gather(track_v, [mid])
                    pred = tm < t16
                    lo = jnp.where(pred, mid + 1, lo)
                    hi = jnp.where(pred, hi, mid)
                idx = jnp.minimum(lo, T_src - 2)
                tl = plsc.load_gather(track_v, [idx])
                tr = plsc.load_gather(track_v, [idx + 1])
                w = jnp.clip((t16 - tl) / (tr - tl + 1e-8), 0.0, 1.0)
                idxL_v[pl.ds(i * 16, 16)] = idx + base_row
                if idxR_v is not None:
                    idxR_v[pl.ds(i * 16, 16)] = idx + 1 + base_row
                w_v[pl.ds(i * 16, 16)] = w
                return carry
            lax.fori_loop(0, T // 16, chunk, 0)

        def issue_gathers(k, p):
            pltpu.async_copy(
                vis_f.at[visidxL_v.at[pl.ds(k * CH, CH)]], visL[p], gsem)
            pltpu.async_copy(
                vis_f.at[visidxR_v.at[pl.ds(k * CH, CH)]], visR[p], gsem)

        def drain_gathers(p):
            pltpu.make_async_copy(vis_f.at[pl.ds(0, CH)], visL[p],
                                  gsem).wait()
            pltpu.make_async_copy(vis_f.at[pl.ds(0, CH)], visR[p],
                                  gsem).wait()

        def drain_out(b, p):
            pltpu.make_async_copy(fused[p], out.at[b, pl.ds(0, CH), :],
                                  osem).wait()

        def sample(s, carry):
            b = wid * SPW + s
            pltpu.sync_copy(tgt.at[b], tgt_v)
            pltpu.sync_copy(vis_t.at[b], vist_v)
            pltpu.sync_copy(prop_t.at[b], propt_v)
            pltpu.sync_copy(prop_f.at[b], propfeat_v)
            pltpu.sync_copy(lang_avg.at[b], langsum_v)

            searchsorted(vist_v, T_vis, 8, b * T_vis,
                         visidxL_v, visidxR_v, visw_v)
            searchsorted(propt_v, T_prop, 10, 0,
                         propidx_v, None, propw_v)

            for p in (0, 1):
                def fill(r, carry, _p=p):
                    for c in range(D_lang // 16):
                        fused[_p][r, pl.ds(D_vis + D_prop + c * 16, 16)] = (
                            langsum_v[pl.ds(c * 16, 16)])
                    return carry
                lax.fori_loop(0, CH, fill, 0)

            lane = lax.iota(jnp.int32, 16)

            def process(k, p):
                @pl.when(k + 1 < NCH)
                def _():
                    issue_gathers(k + 1, 1 - p)
                drain_gathers(p)

                @pl.when(k >= 2)
                def _():
                    drain_out(b, p)

                fz = fused[p]
                vL = visL[p]
                vR = visR[p]

                @plsc.parallel_loop(0, CH, unroll=2)
                def _(r):
                    g = _splat16(k * CH + r)
                    wv = plsc.load_gather(visw_v, [g])
                    for c in range(D_vis // 16):
                        sl = pl.ds(c * 16, 16)
                        lv = vL[r, sl]
                        rv = vR[r, sl]
                        fz[r, sl] = lv + wv * (rv - lv)
                    wp = plsc.load_gather(propw_v, [g])
                    pi = plsc.load_gather(propidx_v, [g])
                    for c in range(D_prop // 16):
                        col = lane + c * 16
                        lv = plsc.load_gather(propfeat_v, [pi, col])
                        rv = plsc.load_gather(propfeat_v, [pi + 1, col])
                        fz[r, pl.ds(D_vis + c * 16, 16)] = (
                            lv + wp * (rv - lv))

                pltpu.async_copy(fz, out.at[b, pl.ds(k * CH, CH), :], osem)

            issue_gathers(0, 0)

            def chunk_pair(j, carry):
                process(2 * j, 0)
                process(2 * j + 1, 1)
                return carry
            lax.fori_loop(0, NCH // 2, chunk_pair, 0)

            drain_out(b, 0)
            drain_out(b, 1)
            return carry

        lax.fori_loop(0, SPW, sample, 0)

    mesh = plsc.VectorSubcoreMesh(core_axis_name="c", subcore_axis_name="s")
    return pl.kernel(
        body,
        out_type=jax.ShapeDtypeStruct((B, T, D_out), jnp.float32),
        mesh=mesh,
        compiler_params=pltpu.CompilerParams(needs_layout_passes=False),
        scratch_types=[
            pltpu.VMEM((T,), jnp.float32),
            pltpu.VMEM((T_vis,), jnp.float32),
            pltpu.VMEM((T_prop,), jnp.float32),
            pltpu.VMEM((T,), jnp.int32),
            pltpu.VMEM((T,), jnp.int32),
            pltpu.VMEM((T,), jnp.float32),
            pltpu.VMEM((T,), jnp.int32),
            pltpu.VMEM((T,), jnp.float32),
            pltpu.VMEM((T_prop, D_prop), jnp.float32),
            pltpu.VMEM((D_lang,), jnp.float32),
            pltpu.VMEM((CH, D_out), jnp.float32),
            pltpu.VMEM((CH, D_out), jnp.float32),
            pltpu.VMEM((CH, D_vis), jnp.float32),
            pltpu.VMEM((CH, D_vis), jnp.float32),
            pltpu.VMEM((CH, D_vis), jnp.float32),
            pltpu.VMEM((CH, D_vis), jnp.float32),
            pltpu.SemaphoreType.DMA,
            pltpu.SemaphoreType.DMA,
        ],
    )


def kernel(vision_features, vision_timestamps, proprio_features,
           proprio_timestamps, lang_embeddings, target_timestamps):
    B, T_vis, D_vis = vision_features.shape
    _, T_prop, D_prop = proprio_features.shape
    _, L, D_lang = lang_embeddings.shape
    T = target_timestamps.shape[1]

    lang_avg = _lang_mean(lang_embeddings)
    k = _make_sc_kernel(B, T, T_vis, D_vis, T_prop, D_prop, D_lang)
    return k(vision_features.reshape(B * T_vis, D_vis), vision_timestamps,
             proprio_features, proprio_timestamps, lang_avg,
             target_timestamps)
